# baseline (speedup 1.0000x reference)
import functools

import jax
import jax.numpy as jnp
from jax import lax
from jax.experimental import pallas as pl
from jax.experimental.pallas import tpu as pltpu
from jax.experimental.pallas import tpu_sc as plsc

NC = 2
NS = 16
NW = NC * NS
LANES = 16

VOCAB = 1000000
EMBED = 64
BATCH = 4096
CTX = 20

RPW = BATCH // NW
CHUNK = 32
NCHUNK = RPW // CHUNK
GPC = CHUNK * CTX // 128
EV = EMBED // LANES


def _cbow_body(w_in, w_out, ctx_idx, tgt_idx, out, idx_v, tidx_v,
               ctx_rows, tgt_rows, scores_v, sem):
  wid = lax.axis_index("s") * NC + lax.axis_index("c")

  pltpu.sync_copy(tgt_idx.at[wid], tidx_v)
  pltpu.async_copy(w_out.at[tidx_v], tgt_rows, sem).wait()

  inv_ctx = jnp.float32(1.0 / CTX)
  lane_iota = lax.iota(jnp.int32, LANES)

  for chunk in range(NCHUNK):
    pltpu.sync_copy(ctx_idx.at[wid, chunk], idx_v)
    copies = []
    for j in range(GPC):
      copies.append(
          pltpu.async_copy(w_in.at[idx_v.at[j]],
                           ctx_rows.at[pl.ds(j * 128, 128)], sem))
    for c in copies:
      c.wait()

    def row_body(r, svec, chunk=chunk):
      base = r * CTX
      trow = chunk * CHUNK + r
      prod = None
      for e in range(EV):
        acc = ctx_rows[base, pl.ds(e * LANES, LANES)]
        for c in range(1, CTX):
          acc = acc + ctx_rows[base + c, pl.ds(e * LANES, LANES)]
        term = acc * tgt_rows[trow, pl.ds(e * LANES, LANES)]
        prod = term if prod is None else prod + term
      s = jnp.sum(prod) * inv_ctx
      svec = jnp.where(lane_iota == (r & (LANES - 1)), s, svec)

      @pl.when((r & (LANES - 1)) == LANES - 1)
      def _store(svec=svec, r=r):
        scores_v[pl.ds(chunk * CHUNK + (r & ~(LANES - 1)), LANES)] = svec

      return svec

    lax.fori_loop(0, CHUNK, row_body, jnp.zeros((LANES,), jnp.float32))

  pltpu.sync_copy(scores_v, out.at[wid])


@jax.jit
def _cbow(ctx_idx, tgt_idx, w_in, w_out):
  mesh = plsc.VectorSubcoreMesh(core_axis_name="c", subcore_axis_name="s")
  f = pl.kernel(
      _cbow_body,
      out_type=jax.ShapeDtypeStruct((NW, RPW), jnp.float32),
      mesh=mesh,
      compiler_params=pltpu.CompilerParams(
          needs_layout_passes=False, use_tc_tiling_on_sc=False),
      scratch_types=[
          pltpu.VMEM((GPC, 128), jnp.int32),
          pltpu.VMEM((RPW,), jnp.int32),
          pltpu.VMEM((CHUNK * CTX, EMBED), jnp.float32),
          pltpu.VMEM((RPW, EMBED), jnp.float32),
          pltpu.VMEM((RPW,), jnp.float32),
          pltpu.SemaphoreType.DMA,
      ],
  )
  return f(w_in, w_out, ctx_idx, tgt_idx)


def kernel(context_ids, target_ids, W_in, W_out):
  ctx_idx = context_ids.astype(jnp.int32).reshape(NW, NCHUNK, GPC, 128)
  tgt_idx = target_ids.astype(jnp.int32).reshape(NW, RPW)
  out = _cbow(ctx_idx, tgt_idx, W_in, W_out)
  return out.reshape(BATCH)


# speedup vs baseline: 1.6999x; 1.6999x over previous
import functools

import jax
import jax.numpy as jnp
from jax import lax
from jax.experimental import pallas as pl
from jax.experimental.pallas import tpu as pltpu
from jax.experimental.pallas import tpu_sc as plsc

NC = 2
NS = 16
NW = NC * NS
LANES = 16

VOCAB = 1000000
EMBED = 64
BATCH = 4096
CTX = 20

RPW = BATCH // NW
CHUNK = 32
NCHUNK = RPW // CHUNK
GPC = CHUNK * CTX // 128
EV = EMBED // LANES
NBUF = 8
KT = EMBED // 8


def _tgt_body(wt3, tgt_idx, out, tidx_v, stage, out_v, sems):
  wid = lax.axis_index("s") * NC + lax.axis_index("c")
  pltpu.sync_copy(tgt_idx.at[wid], tidx_v.at[pl.ds(0, RPW)])

  lane = lax.iota(jnp.int32, LANES)
  i0 = []
  i1 = []
  for ev in range(EV):
    e = ev * LANES + lane
    i0.append(e >> 3)
    i1.append(e & 7)

  def scal(v, l):
    return jnp.sum(jnp.where(lane == l, v, jnp.int32(0)))

  def issue(col, slot):
    col = pl.multiple_of(col, 128)
    pltpu.async_copy(
        wt3.at[:, :, pl.ds(col, 128)], stage.at[slot], sems.at[slot])

  def drain(slot):
    pltpu.make_async_copy(
        wt3.at[:, :, pl.ds(0, 128)], stage.at[slot], sems.at[slot]).wait()

  idvec0 = tidx_v[pl.ds(0, LANES)]
  blk0 = (idvec0 >> 7) * 128
  for slot in range(NBUF):
    issue(scal(blk0, slot), slot)

  def phase_body(ph, _):
    idcur = tidx_v[pl.ds(ph * NBUF, LANES)]
    civ = idcur & 127
    idnext = tidx_v[pl.ds(ph * NBUF + NBUF, LANES)]
    blknext = (idnext >> 7) * 128
    for slot in range(NBUF):
      r = ph * NBUF + slot
      drain(slot)
      ci = jnp.full((LANES,), scal(civ, slot), jnp.int32)
      for ev in range(EV):
        row = plsc.load_gather(stage.at[slot], [i0[ev], i1[ev], ci])
        out_v[r, pl.ds(ev * LANES, LANES)] = row

      @pl.when(ph < RPW // NBUF - 1)
      def _next(slot=slot, blknext=blknext):
        issue(scal(blknext, slot), slot)

    return 0

  lax.fori_loop(0, RPW // NBUF, phase_body, 0)
  pltpu.sync_copy(out_v, out.at[wid])


def _cbow_body(w_in, ctx_idx, tgt_hbm, out, idx_v, ctx_rows, tgt_rows,
               scores_v, sem):
  wid = lax.axis_index("s") * NC + lax.axis_index("c")

  pltpu.sync_copy(tgt_hbm.at[wid], tgt_rows)

  inv_ctx = jnp.float32(1.0 / CTX)
  lane_iota = lax.iota(jnp.int32, LANES)

  for chunk in range(NCHUNK):
    pltpu.sync_copy(ctx_idx.at[wid, chunk], idx_v)
    copies = []
    for j in range(GPC):
      copies.append(
          pltpu.async_copy(w_in.at[idx_v.at[j]],
                           ctx_rows.at[pl.ds(j * 128, 128)], sem))
    for c in copies:
      c.wait()

    def row_body(r, svec, chunk=chunk):
      base = r * CTX
      trow = chunk * CHUNK + r
      prod = None
      for e in range(EV):
        acc = ctx_rows[base, pl.ds(e * LANES, LANES)]
        for c in range(1, CTX):
          acc = acc + ctx_rows[base + c, pl.ds(e * LANES, LANES)]
        term = acc * tgt_rows[trow, pl.ds(e * LANES, LANES)]
        prod = term if prod is None else prod + term
      s = jnp.sum(prod) * inv_ctx
      svec = jnp.where(lane_iota == (r & (LANES - 1)), s, svec)

      @pl.when((r & (LANES - 1)) == LANES - 1)
      def _store(svec=svec, r=r):
        scores_v[pl.ds(chunk * CHUNK + (r & ~(LANES - 1)), LANES)] = svec

      return svec

    lax.fori_loop(0, CHUNK, row_body, jnp.zeros((LANES,), jnp.float32))

  pltpu.sync_copy(scores_v, out.at[wid])


@jax.jit
def _cbow(ctx_idx, tgt_idx, w_in, w_out):
  mesh = plsc.VectorSubcoreMesh(core_axis_name="c", subcore_axis_name="s")

  wt3 = w_out.T.reshape(KT, 8, VOCAB)
  tgt_fn = pl.kernel(
      _tgt_body,
      out_type=jax.ShapeDtypeStruct((NW, RPW, EMBED), jnp.float32),
      mesh=mesh,
      compiler_params=pltpu.CompilerParams(needs_layout_passes=False),
      scratch_types=[
          pltpu.VMEM((RPW + LANES,), jnp.int32),
          pltpu.VMEM((NBUF, KT, 8, 128), jnp.float32),
          pltpu.VMEM((RPW, EMBED), jnp.float32),
          pltpu.SemaphoreType.DMA((NBUF,)),
      ],
  )
  tgt_rows = tgt_fn(wt3, tgt_idx)

  cbow_fn = pl.kernel(
      _cbow_body,
      out_type=jax.ShapeDtypeStruct((NW, RPW), jnp.float32),
      mesh=mesh,
      compiler_params=pltpu.CompilerParams(
          needs_layout_passes=False, use_tc_tiling_on_sc=False),
      scratch_types=[
          pltpu.VMEM((GPC, 128), jnp.int32),
          pltpu.VMEM((CHUNK * CTX, EMBED), jnp.float32),
          pltpu.VMEM((RPW, EMBED), jnp.float32),
          pltpu.VMEM((RPW,), jnp.float32),
          pltpu.SemaphoreType.DMA,
      ],
  )
  return cbow_fn(w_in, ctx_idx, tgt_rows)


def kernel(context_ids, target_ids, W_in, W_out):
  ctx_idx = context_ids.astype(jnp.int32).reshape(NW, NCHUNK, GPC, 128)
  tgt_idx = target_ids.astype(jnp.int32).reshape(NW, RPW)
  out = _cbow(ctx_idx, tgt_idx, W_in, W_out)
  return out.reshape(BATCH)
